# Initial kernel scaffold; baseline (speedup 1.0000x reference)
import jax
import jax.numpy as jnp
from jax.experimental import pallas as pl


def kernel(x, target):
    raise NotImplementedError("write your pallas kernel here")


# speedup vs baseline: 4.9308x; 4.9308x over previous
import math

import jax
import jax.numpy as jnp
from jax import lax
from jax.experimental import pallas as pl
from jax.experimental.pallas import tpu as pltpu
from jax.experimental.pallas import tpu_sc as plsc

_PAD = 0
_SMOOTH = 0.1
_CONF = 1.0 - _SMOOTH

_NC, _NS, _L = 2, 16, 16
_NW = _NC * _NS

_BR = 256


def _tc_body(tgt_ref, x_ref, out_ref):
    i = pl.program_id(0)

    @pl.when(i == 0)
    def _init():
        out_ref[0, 0] = 0.0
        out_ref[0, 1] = 0.0

    xb = x_ref[...]
    w = (tgt_ref[0, 0, :] != _PAD).astype(jnp.float32)
    sx = jnp.sum(xb * w[:, None]) - jnp.sum(xb[:, 0] * w)
    out_ref[0, 0] += sx
    out_ref[0, 1] += jnp.sum(w)


def _tc_masked_sum(x2, tgt3):
    bs, v = x2.shape
    return pl.pallas_call(
        _tc_body,
        grid=(bs // _BR,),
        in_specs=[
            pl.BlockSpec((1, 1, _BR), lambda i: (i, 0, 0)),
            pl.BlockSpec((_BR, v), lambda i: (i, 0)),
        ],
        out_specs=pl.BlockSpec(memory_space=pltpu.SMEM),
        out_shape=jax.ShapeDtypeStruct((1, 2), jnp.float32),
    )(tgt3, x2)


def _sc_corrections(xf, tgtf, b_, s_, v_):
    vchunk = v_ // _NW
    nvec = 2 * b_ * _L
    mesh = plsc.VectorSubcoreMesh(core_axis_name="c", subcore_axis_name="s")

    def body(xf_hbm, tgt_hbm, out_hbm, tgt_v, pres_v, x0_v, part_v):
        wid = lax.axis_index("s") * _NC + lax.axis_index("c")
        lo = wid * vchunk
        hi = lo + vchunk
        pltpu.sync_copy(tgt_hbm, tgt_v)
        for b in range(b_):
            pltpu.sync_copy(
                xf_hbm.at[pl.ds(b * s_ * v_ + lo, vchunk)],
                x0_v.at[pl.ds(b * vchunk, vchunk)],
            )

        def zero_body(i, c):
            pres_v[pl.ds(i * _L, _L)] = jnp.zeros((_L,), jnp.float32)
            return c

        lax.fori_loop(0, (b_ * vchunk) // _L, zero_body, 0)

        ones = jnp.full((_L,), 1.0, jnp.float32)
        for b in range(b_):
            def scat_body(i, c, b=b):
                tgt = tgt_v[pl.ds(b * s_ + i * _L, _L)]
                m = (tgt >= lo) & (tgt < hi) & (tgt != _PAD)
                plsc.store_scatter(
                    pres_v, [tgt - lo + (b * vchunk)], ones, mask=m)
                return c

            lax.fori_loop(0, s_ // _L, scat_body, 0)

        for b in range(b_):
            def red_body(i, acc, b=b):
                p = pres_v[pl.ds(b * vchunk + i * _L, _L)]
                q = x0_v[pl.ds(b * vchunk + i * _L, _L)]
                return acc[0] + p, acc[1] + p * q

            acc_k, acc_d = lax.fori_loop(
                0, vchunk // _L, red_body,
                (jnp.zeros((_L,), jnp.float32), jnp.zeros((_L,), jnp.float32)))
            part_v[pl.ds(b * _L, _L)] = acc_k
            part_v[pl.ds((b_ + b) * _L, _L)] = acc_d

        pltpu.sync_copy(part_v, out_hbm.at[wid])

    call = pl.kernel(
        body,
        out_type=jax.ShapeDtypeStruct((_NW, nvec), jnp.float32),
        mesh=mesh,
        compiler_params=pltpu.CompilerParams(needs_layout_passes=False),
        scratch_types=[
            pltpu.VMEM((b_ * s_,), jnp.int32),
            pltpu.VMEM((b_ * vchunk,), jnp.float32),
            pltpu.VMEM((b_ * vchunk,), jnp.float32),
            pltpu.VMEM((nvec,), jnp.float32),
        ],
    )
    return call(xf, tgtf)


def kernel(x, target):
    b_, s_, v_ = x.shape
    bs = b_ * s_
    tgt = target.astype(jnp.int32)
    x2 = x.reshape(bs, v_)

    main = _tc_masked_sum(x2, tgt.reshape(bs // _BR, 1, _BR))
    sc_out = _sc_corrections(x.reshape(-1), tgt.reshape(-1), b_, s_, v_)

    parts = jnp.sum(sc_out.reshape(_NW, 2 * b_, _L), axis=(0, 2))
    k_b = parts[:b_]
    d_b = parts[b_:]

    f = _SMOOTH / (s_ - 2)
    logf = math.log(f)
    logc = math.log(_CONF)
    m0 = (tgt[:, 0] != _PAD).astype(jnp.float32)
    corr = jnp.sum(m0 * (k_b * (_CONF * logc - f * logf) - (_CONF - f) * d_b))
    return f * ((v_ - 1) * logf * main[0, 1] - main[0, 0]) + corr


# speedup vs baseline: 12.1449x; 2.4631x over previous
import math

import jax
import jax.numpy as jnp
from jax import lax
from jax.experimental import pallas as pl
from jax.experimental.pallas import tpu as pltpu
from jax.experimental.pallas import tpu_sc as plsc

_PAD = 0
_SMOOTH = 0.1
_CONF = 1.0 - _SMOOTH

_NC, _NS, _L = 2, 16, 16
_NW = _NC * _NS

_BR = 256


def _tc_body(tgt_ref, x_ref, out_ref):
    i = pl.program_id(0)

    @pl.when(i == 0)
    def _init():
        out_ref[0, 0] = 0.0
        out_ref[0, 1] = 0.0

    xb = x_ref[...]
    w = (tgt_ref[0, 0, :] != _PAD).astype(jnp.float32)
    sx = jnp.sum(xb * w[:, None]) - jnp.sum(xb[:, 0] * w)
    out_ref[0, 0] += sx
    out_ref[0, 1] += jnp.sum(w)


def _tc_masked_sum(x2, tgt3):
    bs, v = x2.shape
    return pl.pallas_call(
        _tc_body,
        grid=(bs // _BR,),
        in_specs=[
            pl.BlockSpec((1, 1, _BR), lambda i: (i, 0, 0)),
            pl.BlockSpec((_BR, v), lambda i: (i, 0)),
        ],
        out_specs=pl.BlockSpec(memory_space=pltpu.SMEM),
        out_shape=jax.ShapeDtypeStruct((1, 2), jnp.float32),
    )(tgt3, x2)


def _sc_corrections(x2, tgtf, b_, s_, v_):
    vchunk = v_ // _NW
    nvec = 2 * b_ * _L
    mesh = plsc.VectorSubcoreMesh(core_axis_name="c", subcore_axis_name="s")

    def body(xf_hbm, tgt_hbm, out_hbm, tgt_v, pres_v, x0_v, part_v):
        wid = lax.axis_index("s") * _NC + lax.axis_index("c")
        lo = wid * vchunk
        hi = lo + vchunk
        pltpu.sync_copy(tgt_hbm, tgt_v)
        for b in range(b_):
            pltpu.sync_copy(
                xf_hbm.at[b * s_, pl.ds(lo, vchunk)],
                x0_v.at[pl.ds(b * vchunk, vchunk)],
            )

        def zero_body(i, c):
            pres_v[pl.ds(i * _L, _L)] = jnp.zeros((_L,), jnp.float32)
            return c

        lax.fori_loop(0, (b_ * vchunk) // _L, zero_body, 0)

        ones = jnp.full((_L,), 1.0, jnp.float32)
        for b in range(b_):
            def scat_body(i, c, b=b):
                tgt = tgt_v[pl.ds(b * s_ + i * _L, _L)]
                m = (tgt >= lo) & (tgt < hi) & (tgt != _PAD)
                plsc.store_scatter(
                    pres_v, [tgt - lo + (b * vchunk)], ones, mask=m)
                return c

            lax.fori_loop(0, s_ // _L, scat_body, 0)

        for b in range(b_):
            def red_body(i, acc, b=b):
                p = pres_v[pl.ds(b * vchunk + i * _L, _L)]
                q = x0_v[pl.ds(b * vchunk + i * _L, _L)]
                return acc[0] + p, acc[1] + p * q

            acc_k, acc_d = lax.fori_loop(
                0, vchunk // _L, red_body,
                (jnp.zeros((_L,), jnp.float32), jnp.zeros((_L,), jnp.float32)))
            part_v[pl.ds(b * _L, _L)] = acc_k
            part_v[pl.ds((b_ + b) * _L, _L)] = acc_d

        pltpu.sync_copy(part_v, out_hbm.at[wid])

    call = pl.kernel(
        body,
        out_type=jax.ShapeDtypeStruct((_NW, nvec), jnp.float32),
        mesh=mesh,
        compiler_params=pltpu.CompilerParams(needs_layout_passes=False),
        scratch_types=[
            pltpu.VMEM((b_ * s_,), jnp.int32),
            pltpu.VMEM((b_ * vchunk,), jnp.float32),
            pltpu.VMEM((b_ * vchunk,), jnp.float32),
            pltpu.VMEM((nvec,), jnp.float32),
        ],
    )
    return call(x2, tgtf)


def kernel(x, target):
    b_, s_, v_ = x.shape
    bs = b_ * s_
    tgt = target.astype(jnp.int32)
    x2 = x.reshape(bs, v_)

    main = _tc_masked_sum(x2, tgt.reshape(bs // _BR, 1, _BR))
    sc_out = _sc_corrections(x2, tgt.reshape(-1), b_, s_, v_)

    parts = jnp.sum(sc_out.reshape(_NW, 2 * b_, _L), axis=(0, 2))
    k_b = parts[:b_]
    d_b = parts[b_:]

    f = _SMOOTH / (s_ - 2)
    logf = math.log(f)
    logc = math.log(_CONF)
    m0 = (tgt[:, 0] != _PAD).astype(jnp.float32)
    corr = jnp.sum(m0 * (k_b * (_CONF * logc - f * logf) - (_CONF - f) * d_b))
    return f * ((v_ - 1) * logf * main[0, 1] - main[0, 0]) + corr


# speedup vs baseline: 13.2020x; 1.0870x over previous
import math

import jax
import jax.numpy as jnp
from jax import lax
from jax.experimental import pallas as pl
from jax.experimental.pallas import tpu as pltpu
from jax.experimental.pallas import tpu_sc as plsc

_PAD = 0
_SMOOTH = 0.1
_CONF = 1.0 - _SMOOTH

_NC, _NS, _L = 2, 16, 16
_NW = _NC * _NS

_BR = 512


def _tc_body(tgt_ref, x_ref, out_ref):
    i = pl.program_id(0)

    @pl.when(i == 0)
    def _init():
        out_ref[0, 0] = 0.0
        out_ref[0, 1] = 0.0

    xb = x_ref[...]
    w = (tgt_ref[0, 0, :] != _PAD).astype(jnp.float32)
    sx = jnp.sum(xb * w[:, None]) - jnp.sum(xb[:, 0] * w)
    out_ref[0, 0] += sx
    out_ref[0, 1] += jnp.sum(w)


def _tc_masked_sum(x2, tgt3):
    bs, v = x2.shape
    return pl.pallas_call(
        _tc_body,
        grid=(bs // _BR,),
        in_specs=[
            pl.BlockSpec((1, 1, _BR), lambda i: (i, 0, 0)),
            pl.BlockSpec((_BR, v), lambda i: (i, 0)),
        ],
        out_specs=pl.BlockSpec(memory_space=pltpu.SMEM),
        out_shape=jax.ShapeDtypeStruct((1, 2), jnp.float32),
    )(tgt3, x2)


def _sc_corrections(x2, tgtf, b_, s_, v_):
    vchunk = v_ // _NW
    nvec = 2 * b_ * _L
    mesh = plsc.VectorSubcoreMesh(core_axis_name="c", subcore_axis_name="s")

    def body(xf_hbm, tgt_hbm, out_hbm, tgt_v, pres_v, x0_v, part_v):
        wid = lax.axis_index("s") * _NC + lax.axis_index("c")
        lo = wid * vchunk
        hi = lo + vchunk
        pltpu.sync_copy(tgt_hbm, tgt_v)
        for b in range(b_):
            pltpu.sync_copy(
                xf_hbm.at[b * s_, pl.ds(lo, vchunk)],
                x0_v.at[pl.ds(b * vchunk, vchunk)],
            )

        def zero_body(i, c):
            pres_v[pl.ds(i * _L, _L)] = jnp.zeros((_L,), jnp.float32)
            return c

        lax.fori_loop(0, (b_ * vchunk) // _L, zero_body, 0)

        ones = jnp.full((_L,), 1.0, jnp.float32)
        for b in range(b_):
            def scat_body(i, c, b=b):
                tgt = tgt_v[pl.ds(b * s_ + i * _L, _L)]
                m = (tgt >= lo) & (tgt < hi) & (tgt != _PAD)
                plsc.store_scatter(
                    pres_v, [tgt - lo + (b * vchunk)], ones, mask=m)
                return c

            lax.fori_loop(0, s_ // _L, scat_body, 0)

        for b in range(b_):
            def red_body(i, acc, b=b):
                p = pres_v[pl.ds(b * vchunk + i * _L, _L)]
                q = x0_v[pl.ds(b * vchunk + i * _L, _L)]
                return acc[0] + p, acc[1] + p * q

            acc_k, acc_d = lax.fori_loop(
                0, vchunk // _L, red_body,
                (jnp.zeros((_L,), jnp.float32), jnp.zeros((_L,), jnp.float32)))
            part_v[pl.ds(b * _L, _L)] = acc_k
            part_v[pl.ds((b_ + b) * _L, _L)] = acc_d

        pltpu.sync_copy(part_v, out_hbm.at[wid])

    call = pl.kernel(
        body,
        out_type=jax.ShapeDtypeStruct((_NW, nvec), jnp.float32),
        mesh=mesh,
        compiler_params=pltpu.CompilerParams(needs_layout_passes=False),
        scratch_types=[
            pltpu.VMEM((b_ * s_,), jnp.int32),
            pltpu.VMEM((b_ * vchunk,), jnp.float32),
            pltpu.VMEM((b_ * vchunk,), jnp.float32),
            pltpu.VMEM((nvec,), jnp.float32),
        ],
    )
    return call(x2, tgtf)


def kernel(x, target):
    b_, s_, v_ = x.shape
    bs = b_ * s_
    tgt = target.astype(jnp.int32)
    x2 = x.reshape(bs, v_)

    main = _tc_masked_sum(x2, tgt.reshape(bs // _BR, 1, _BR))
    sc_out = _sc_corrections(x2, tgt.reshape(-1), b_, s_, v_)

    parts = jnp.sum(sc_out.reshape(_NW, 2 * b_, _L), axis=(0, 2))
    k_b = parts[:b_]
    d_b = parts[b_:]

    f = _SMOOTH / (s_ - 2)
    logf = math.log(f)
    logc = math.log(_CONF)
    m0 = (tgt[:, 0] != _PAD).astype(jnp.float32)
    corr = jnp.sum(m0 * (k_b * (_CONF * logc - f * logf) - (_CONF - f) * d_b))
    return f * ((v_ - 1) * logf * main[0, 1] - main[0, 0]) + corr


# speedup vs baseline: 14.7175x; 1.1148x over previous
import math

import jax
import jax.numpy as jnp
from jax import lax
from jax.experimental import pallas as pl
from jax.experimental.pallas import tpu as pltpu
from jax.experimental.pallas import tpu_sc as plsc

_PAD = 0
_SMOOTH = 0.1
_CONF = 1.0 - _SMOOTH

_NC, _NS, _L = 2, 16, 16
_NW = _NC * _NS

_BR = 128


_NSTREAM = 4


def _tc_masked_sum(x2, tgt):
    bs, v = x2.shape
    b_, s_ = tgt.shape
    part = bs // (_NSTREAM * _BR)
    assert _NSTREAM == b_ and part * _BR == s_
    f = _SMOOTH / (s_ - 2)
    logf = math.log(f)

    def body(*refs):
        out_ref = refs[-1]
        i = pl.program_id(0)

        @pl.when(i == 0)
        def _init():
            out_ref[0, 0] = 0.0
            out_ref[0, 1] = 0.0

        tgt_ref = refs[0]
        sx = 0.0
        nv = 0.0
        for k in range(_NSTREAM):
            x_ref = refs[1 + k]
            xb = x_ref[...]
            w = (tgt_ref[k, pl.ds(i * _BR, _BR)] != _PAD).astype(jnp.float32)
            sx += jnp.sum(xb * w[:, None]) - jnp.sum(xb[:, 0] * w)
            nv += jnp.sum(w)
        out_ref[0, 0] += sx
        out_ref[0, 1] += nv

        @pl.when(i == part - 1)
        def _finalize():
            out_ref[0, 0] = f * (
                (v - 1) * logf * out_ref[0, 1] - out_ref[0, 0])

    def xmap(k):
        return lambda i: (i + k * part, 0)

    return pl.pallas_call(
        body,
        grid=(part,),
        in_specs=[pl.BlockSpec((b_, s_), lambda i: (0, 0))]
        + [pl.BlockSpec((_BR, v), xmap(k)) for k in range(_NSTREAM)],
        out_specs=pl.BlockSpec(memory_space=pltpu.SMEM),
        out_shape=jax.ShapeDtypeStruct((1, 2), jnp.float32),
    )(*([tgt] + [x2] * _NSTREAM))


def _sc_corrections(x2, tgt, b_, s_, v_):
    vchunk = v_ // _NW
    f = _SMOOTH / (s_ - 2)
    coef_k = _CONF * math.log(_CONF) - f * math.log(f)
    coef_d = f - _CONF
    mesh = plsc.VectorSubcoreMesh(core_axis_name="c", subcore_axis_name="s")

    def body(xf_hbm, tgt_hbm, out_hbm, tgt_v, pres_v, x0_v, part_v):
        wid = lax.axis_index("s") * _NC + lax.axis_index("c")
        lo = wid * vchunk
        hi = lo + vchunk
        pltpu.sync_copy(tgt_hbm, tgt_v)
        for b in range(b_):
            pltpu.sync_copy(
                xf_hbm.at[b * s_, pl.ds(lo, vchunk)],
                x0_v.at[pl.ds(b * vchunk, vchunk)],
            )

        def zero_body(i, c):
            pres_v[pl.ds(i * _L, _L)] = jnp.zeros((_L,), jnp.float32)
            return c

        lax.fori_loop(0, (b_ * vchunk) // _L, zero_body, 0)

        ones = jnp.full((_L,), 1.0, jnp.float32)
        for b in range(b_):
            def scat_body(i, c, b=b):
                t16 = tgt_v[b, pl.ds(i * _L, _L)]
                m = (t16 >= lo) & (t16 < hi) & (t16 != _PAD)
                plsc.store_scatter(
                    pres_v, [t16 - lo + (b * vchunk)], ones, mask=m)
                return c

            lax.fori_loop(0, s_ // _L, scat_body, 0)

        final = jnp.zeros((_L,), jnp.float32)
        for b in range(b_):
            def red_body(i, acc, b=b):
                p = pres_v[pl.ds(b * vchunk + i * _L, _L)]
                q = x0_v[pl.ds(b * vchunk + i * _L, _L)]
                return acc[0] + p, acc[1] + p * q

            acc_k, acc_d = lax.fori_loop(
                0, vchunk // _L, red_body,
                (jnp.zeros((_L,), jnp.float32), jnp.zeros((_L,), jnp.float32)))
            gate = jnp.where(tgt_v[b, pl.ds(0, _L)][0] != _PAD, 1.0, 0.0)
            final = final + gate * (coef_k * acc_k + coef_d * acc_d)

        part_v[...] = final
        pltpu.sync_copy(part_v, out_hbm.at[wid])

    call = pl.kernel(
        body,
        out_type=jax.ShapeDtypeStruct((_NW, _L), jnp.float32),
        mesh=mesh,
        compiler_params=pltpu.CompilerParams(needs_layout_passes=False),
        scratch_types=[
            pltpu.VMEM((b_, s_), jnp.int32),
            pltpu.VMEM((b_ * vchunk,), jnp.float32),
            pltpu.VMEM((b_ * vchunk,), jnp.float32),
            pltpu.VMEM((_L,), jnp.float32),
        ],
    )
    return call(x2, tgt)


def kernel(x, target):
    b_, s_, v_ = x.shape
    bs = b_ * s_
    tgt = target.astype(jnp.int32)
    x2 = x.reshape(bs, v_)

    main = _tc_masked_sum(x2, tgt)
    sc_out = _sc_corrections(x2, tgt, b_, s_, v_)
    return main[0, 0] + jnp.sum(sc_out)
